# baseline (speedup 1.0000x reference)
import functools

import jax
import jax.numpy as jnp
from jax import lax
from jax.experimental import pallas as pl
from jax.experimental.pallas import tpu as pltpu
from jax.experimental.pallas import tpu_sc as plsc

B, S, D = 1, 2048, 1024
F = 4096
E = 8
TOPK = 2
T = B * S
PATCH_K = 3 * 16 * 16
FB = 1024
NF = F // FB
TM = 256
NT = (2 * T) // TM + E + 1
PBUF = NT * TM
TRASH = PBUF - 1
NW = 32
TPW = T // NW


def _router_body(xf_ref, wr_ref, br_ref, rwp_ref, dst_ref, te_ref, aux_ref):
    xf = xf_ref[...]
    logits = jnp.dot(xf, wr_ref[...], preferred_element_type=jnp.float32)
    logits = logits + br_ref[...]
    lmax = jnp.max(logits, axis=1, keepdims=True)
    ex = jnp.exp(logits - lmax)
    probs = ex / jnp.sum(ex, axis=1, keepdims=True)

    ei = lax.broadcasted_iota(jnp.int32, (T, E), 1)
    m1 = jnp.max(probs, axis=1, keepdims=True)
    i1 = jnp.min(jnp.where(probs == m1, ei, E), axis=1, keepdims=True)
    probs2 = jnp.where(ei == i1, -1.0, probs)
    m2 = jnp.max(probs2, axis=1, keepdims=True)
    i2 = jnp.min(jnp.where(probs2 == m2, ei, E), axis=1, keepdims=True)
    ssum = m1 + m2
    rw0 = m1 / ssum
    rw1 = m2 / ssum
    oh1 = (ei == i1).astype(jnp.float32)
    oh2 = (ei == i2).astype(jnp.float32)

    counts = jnp.sum(oh1 + oh2, axis=0, keepdims=True)
    frac = counts / jnp.float32(T * TOPK)
    pmean = jnp.mean(probs, axis=0, keepdims=True)
    aux_ref[...] = jnp.reshape(jnp.float32(E) * jnp.sum(frac * pmean), (1, 1))

    rr = lax.broadcasted_iota(jnp.int32, (T, T), 0)
    cc = lax.broadcasted_iota(jnp.int32, (T, T), 1)
    ltri = (cc < rr).astype(jnp.float32)
    c1 = jnp.dot(ltri, oh1, preferred_element_type=jnp.float32)
    c2 = jnp.dot(ltri, oh2, preferred_element_type=jnp.float32)

    pc = jnp.ceil(counts / TM) * TM
    e8a = lax.broadcasted_iota(jnp.int32, (E, E), 0)
    e8b = lax.broadcasted_iota(jnp.int32, (E, E), 1)
    mex = (e8a < e8b).astype(jnp.float32)
    po = jnp.dot(pc, mex, preferred_element_type=jnp.float32)

    rank0 = c1 + c2
    rank1 = rank0 + oh1
    dest0 = jnp.sum(oh1 * (po + rank0), axis=1, keepdims=True)
    dest1 = jnp.sum(oh2 * (po + rank1), axis=1, keepdims=True)
    dp = jnp.where(i1 == 1, dest0, jnp.where(i2 == 1, dest1, jnp.float32(TRASH)))
    dq = jnp.where(i1 == 3, dest0, jnp.where(i2 == 3, dest1, jnp.float32(TRASH)))

    ci = lax.broadcasted_iota(jnp.int32, (T, 8), 1)
    dst_ref[...] = (dest0 * (ci == 0) + dest1 * (ci == 1)
                    + dp * (ci == 2) + dq * (ci == 3)).astype(jnp.int32)
    rwp_ref[...] = rw0 * (ci == 0) + rw1 * (ci == 1)

    mpos = lax.broadcasted_iota(jnp.int32, (1, 128), 1).astype(jnp.float32) * jnp.float32(TM)
    acc = jnp.full((1, 128), 7, jnp.int32)
    for e in range(E):
        po_e = po[0:1, e:e + 1]
        pc_e = pc[0:1, e:e + 1]
        inside = (mpos >= po_e) & (mpos < po_e + pc_e)
        acc = jnp.where(inside, e, acc)
    te_ref[...] = acc


def _router(xf, Wr, br):
    return pl.pallas_call(
        _router_body,
        out_shape=(
            jax.ShapeDtypeStruct((T, 8), jnp.float32),
            jax.ShapeDtypeStruct((T, 8), jnp.int32),
            jax.ShapeDtypeStruct((1, 128), jnp.int32),
            jax.ShapeDtypeStruct((1, 1), jnp.float32),
        ),
    )(xf, Wr, br.reshape(1, E))


def _sc_scatter(xf, qq, pp, srp, d0, d1, dq, dp, dpr2):
    mesh = plsc.VectorSubcoreMesh(core_axis_name="c", subcore_axis_name="s")

    @functools.partial(
        pl.kernel,
        out_type=[
            jax.ShapeDtypeStruct((PBUF, D), jnp.float32),
            jax.ShapeDtypeStruct((PBUF, D), jnp.float32),
            jax.ShapeDtypeStruct((PBUF, PATCH_K), jnp.float32),
            jax.ShapeDtypeStruct((PBUF, 128), jnp.float32),
        ],
        mesh=mesh,
        scratch_types=[
            pltpu.VMEM((TPW, D), jnp.float32),
            pltpu.VMEM((TPW, PATCH_K), jnp.float32),
            pltpu.VMEM((TPW, 128), jnp.float32),
            pltpu.VMEM((TPW,), jnp.int32),
            pltpu.VMEM((TPW,), jnp.int32),
            pltpu.VMEM((TPW,), jnp.int32),
            pltpu.VMEM((TPW,), jnp.int32),
            pltpu.VMEM((TPW,), jnp.int32),
            pltpu.SemaphoreType.DMA,
        ],
    )
    def scat(xf_h, qq_h, pp_h, srp_h, d0_h, d1_h, dq_h, dp_h, dpr_h,
             xs_h, qs_h, ps_h, srs_h,
             bufx, bufp, bufsr, i0, i1, iq, ip, ipr, sem):
        wid = lax.axis_index("s") * 2 + lax.axis_index("c")
        base = wid * TPW
        pltpu.sync_copy(d0_h.at[wid], i0)
        pltpu.sync_copy(d1_h.at[wid], i1)
        pltpu.sync_copy(dq_h.at[wid], iq)
        pltpu.sync_copy(dp_h.at[wid], ip)

        pltpu.sync_copy(xf_h.at[pl.ds(base, TPW)], bufx)
        a = pltpu.async_copy(bufx, xs_h.at[i0], sem)
        b = pltpu.async_copy(bufx, xs_h.at[i1], sem)
        a.wait()
        b.wait()

        pltpu.sync_copy(qq_h.at[pl.ds(base, TPW)], bufx)
        pltpu.async_copy(bufx, qs_h.at[iq], sem).wait()

        pltpu.sync_copy(pp_h.at[pl.ds(base, TPW)], bufp)
        pltpu.async_copy(bufp, ps_h.at[ip], sem).wait()

        for k in range(2):
            pltpu.sync_copy(dpr_h.at[2 * wid + k], ipr)
            pltpu.sync_copy(srp_h.at[pl.ds(2 * base + k * TPW, TPW)], bufsr)
            pltpu.async_copy(bufsr, srs_h.at[ipr], sem).wait()

    return scat(xf, qq, pp, srp, d0, d1, dq, dp, dpr2)


def _gmm_body(te_ref, xs_ref, sr_ref, ps_ref, qs_ref,
              w1_ref, wsm_ref, wp_ref, wq_ref, b1_ref, w2_ref, b2_ref,
              out_ref, h_ref):
    m = pl.program_id(0)
    f = pl.program_id(1)
    e = te_ref[m]

    h_ref[...] = (jnp.dot(xs_ref[...].astype(jnp.bfloat16), w1_ref[0],
                          preferred_element_type=jnp.float32)
                  + jnp.dot(sr_ref[:, 0:16], wsm_ref[0], preferred_element_type=jnp.float32)
                  + b1_ref[0])

    @pl.when(e == 1)
    def _():
        h_ref[...] = h_ref[...] + jnp.dot(ps_ref[...].astype(jnp.bfloat16), wp_ref[...],
                                          preferred_element_type=jnp.float32)

    @pl.when(e == 3)
    def _():
        h_ref[...] = h_ref[...] + jnp.dot(qs_ref[...].astype(jnp.bfloat16), wq_ref[...],
                                          preferred_element_type=jnp.float32)

    hg = jax.nn.gelu(h_ref[...])
    acc = jnp.dot(hg.astype(jnp.bfloat16), w2_ref[0], preferred_element_type=jnp.float32)

    @pl.when(f == 0)
    def _():
        out_ref[...] = jnp.zeros_like(out_ref)

    out_ref[...] += acc

    @pl.when(f == NF - 1)
    def _():
        out_ref[...] = (out_ref[...] + b2_ref[0]) * sr_ref[:, 16:17]


def _gmm(te, xs, sr, ps, qs, W1, Wsm, Wp, Wq, b1, W2, b2):
    grid_spec = pltpu.PrefetchScalarGridSpec(
        num_scalar_prefetch=1,
        grid=(NT, NF),
        in_specs=[
            pl.BlockSpec((TM, D), lambda m, f, te: (m, 0)),
            pl.BlockSpec((TM, 128), lambda m, f, te: (m, 0)),
            pl.BlockSpec((TM, PATCH_K), lambda m, f, te: (m, 0)),
            pl.BlockSpec((TM, D), lambda m, f, te: (m, 0)),
            pl.BlockSpec((1, D, FB), lambda m, f, te: (te[m], 0, f)),
            pl.BlockSpec((1, 16, FB), lambda m, f, te: (te[m], 0, f)),
            pl.BlockSpec((PATCH_K, FB), lambda m, f, te: (0, jnp.where(te[m] == 1, f, 0))),
            pl.BlockSpec((D, FB), lambda m, f, te: (0, jnp.where(te[m] == 3, f, 0))),
            pl.BlockSpec((1, 1, FB), lambda m, f, te: (te[m], 0, f)),
            pl.BlockSpec((1, FB, D), lambda m, f, te: (te[m], f, 0)),
            pl.BlockSpec((1, 1, D), lambda m, f, te: (te[m], 0, 0)),
        ],
        out_specs=pl.BlockSpec((TM, D), lambda m, f, te: (m, 0)),
        scratch_shapes=[pltpu.VMEM((TM, FB), jnp.float32)],
    )
    return pl.pallas_call(
        _gmm_body,
        grid_spec=grid_spec,
        out_shape=jax.ShapeDtypeStruct((PBUF, D), jnp.float32),
    )(te, xs, sr, ps, qs,
      W1.astype(jnp.bfloat16), Wsm, Wp.astype(jnp.bfloat16),
      Wq.astype(jnp.bfloat16),
      b1.reshape(E, 1, F), W2.astype(jnp.bfloat16), b2.reshape(E, 1, D))


def _sc_combine(outs, d0, d1):
    mesh = plsc.VectorSubcoreMesh(core_axis_name="c", subcore_axis_name="s")

    @functools.partial(
        pl.kernel,
        out_type=[
            jax.ShapeDtypeStruct((T, D), jnp.float32),
            jax.ShapeDtypeStruct((T, D), jnp.float32),
        ],
        mesh=mesh,
        scratch_types=[
            pltpu.VMEM((TPW, D), jnp.float32),
            pltpu.VMEM((TPW,), jnp.int32),
            pltpu.SemaphoreType.DMA,
        ],
    )
    def comb(outs_h, d0_h, d1_h, f0_h, f1_h, bufa, i0, sem):
        wid = lax.axis_index("s") * 2 + lax.axis_index("c")
        base = wid * TPW
        pltpu.sync_copy(d0_h.at[wid], i0)
        pltpu.async_copy(outs_h.at[i0], bufa, sem).wait()
        pltpu.sync_copy(bufa, f0_h.at[pl.ds(base, TPW)])
        pltpu.sync_copy(d1_h.at[wid], i0)
        pltpu.async_copy(outs_h.at[i0], bufa, sem).wait()
        pltpu.sync_copy(bufa, f1_h.at[pl.ds(base, TPW)])

    return comb(outs, d0, d1)


def _add_body(a_ref, b_ref, o_ref):
    o_ref[...] = a_ref[...] + b_ref[...]


def _final_add(a, b):
    bm = T // 8
    return pl.pallas_call(
        _add_body,
        grid=(8,),
        in_specs=[pl.BlockSpec((bm, D), lambda i: (i, 0)),
                  pl.BlockSpec((bm, D), lambda i: (i, 0))],
        out_specs=pl.BlockSpec((bm, D), lambda i: (i, 0)),
        out_shape=jax.ShapeDtypeStruct((T, D), jnp.float32),
    )(a, b)


def kernel(x, avg_question_embedding_flat, flow_vectors, raw_patches, frame_deltas,
           Wr, br, W1, b1, W2, b2, Wf, Wp, Wq, Wd):
    xf = x.reshape(T, D)
    pp = raw_patches.reshape(T, PATCH_K)
    qq = avg_question_embedding_flat
    ssm = jnp.concatenate(
        [flow_vectors.reshape(T, 2), frame_deltas.reshape(T, 4),
         jnp.zeros((T, 10), jnp.float32)], axis=1)
    Wsm = jnp.zeros((E, 16, F), jnp.float32)
    Wsm = Wsm.at[0, 0:2].set(Wf)
    Wsm = Wsm.at[4, 2:6].set(Wd)

    rwp, dst, te, aux = _router(xf, Wr, br)

    d0 = dst[:, 0].reshape(NW, TPW)
    d1 = dst[:, 1].reshape(NW, TPW)
    dpm = dst[:, 2].reshape(NW, TPW)
    dqm = dst[:, 3].reshape(NW, TPW)
    dpr2 = dst[:, 0:2].reshape(2 * NW, TPW)
    srp = jnp.concatenate(
        [jnp.repeat(ssm, 2, axis=0),
         rwp[:, 0:2].reshape(2 * T, 1),
         jnp.zeros((2 * T, 111), jnp.float32)], axis=1)

    xs, qs, ps, srs = _sc_scatter(xf, qq, pp, srp, d0, d1, dqm, dpm, dpr2)

    outs = _gmm(te.reshape(128), xs, srs, ps, qs,
                W1, Wsm, Wp, Wq, b1, W2, b2)

    f0, f1 = _sc_combine(outs, d0, d1)
    out = _final_add(f0, f1)
    return out.reshape(B, S, D), aux[0, 0]


# speedup vs baseline: 1.1010x; 1.1010x over previous
import functools

import jax
import jax.numpy as jnp
from jax import lax
from jax.experimental import pallas as pl
from jax.experimental.pallas import tpu as pltpu
from jax.experimental.pallas import tpu_sc as plsc

B, S, D = 1, 2048, 1024
F = 4096
E = 8
TOPK = 2
T = B * S
PATCH_K = 3 * 16 * 16
FB = 1024
NF = F // FB
TM = 256
NT = (2 * T) // TM + E + 1
PBUF = NT * TM
TRASH = PBUF - 1
NW = 32
TPW = T // NW
CH = TPW // 2


def _router_body(xf_ref, wr_ref, br_ref, rwp_ref, dst_ref, te_ref, aux_ref):
    xf = xf_ref[...]
    logits = jnp.dot(xf, wr_ref[...], preferred_element_type=jnp.float32)
    logits = logits + br_ref[...]
    lmax = jnp.max(logits, axis=1, keepdims=True)
    ex = jnp.exp(logits - lmax)
    probs = ex / jnp.sum(ex, axis=1, keepdims=True)

    ei = lax.broadcasted_iota(jnp.int32, (T, E), 1)
    m1 = jnp.max(probs, axis=1, keepdims=True)
    i1 = jnp.min(jnp.where(probs == m1, ei, E), axis=1, keepdims=True)
    probs2 = jnp.where(ei == i1, -1.0, probs)
    m2 = jnp.max(probs2, axis=1, keepdims=True)
    i2 = jnp.min(jnp.where(probs2 == m2, ei, E), axis=1, keepdims=True)
    ssum = m1 + m2
    rw0 = m1 / ssum
    rw1 = m2 / ssum
    oh1 = (ei == i1).astype(jnp.float32)
    oh2 = (ei == i2).astype(jnp.float32)

    counts = jnp.sum(oh1 + oh2, axis=0, keepdims=True)
    frac = counts / jnp.float32(T * TOPK)
    pmean = jnp.mean(probs, axis=0, keepdims=True)
    aux_ref[...] = jnp.reshape(jnp.float32(E) * jnp.sum(frac * pmean), (1, 1))

    rr = lax.broadcasted_iota(jnp.int32, (T, T), 0)
    cc = lax.broadcasted_iota(jnp.int32, (T, T), 1)
    ltri = (cc < rr).astype(jnp.float32)
    c1 = jnp.dot(ltri, oh1, preferred_element_type=jnp.float32)
    c2 = jnp.dot(ltri, oh2, preferred_element_type=jnp.float32)

    pc = jnp.ceil(counts / TM) * TM
    e8a = lax.broadcasted_iota(jnp.int32, (E, E), 0)
    e8b = lax.broadcasted_iota(jnp.int32, (E, E), 1)
    mex = (e8a < e8b).astype(jnp.float32)
    po = jnp.dot(pc, mex, preferred_element_type=jnp.float32)

    rank0 = c1 + c2
    rank1 = rank0 + oh1
    dest0 = jnp.sum(oh1 * (po + rank0), axis=1, keepdims=True)
    dest1 = jnp.sum(oh2 * (po + rank1), axis=1, keepdims=True)
    dp = jnp.where(i1 == 1, dest0, jnp.where(i2 == 1, dest1, jnp.float32(TRASH)))
    dq = jnp.where(i1 == 3, dest0, jnp.where(i2 == 3, dest1, jnp.float32(TRASH)))

    ci = lax.broadcasted_iota(jnp.int32, (T, 8), 1)
    dst_ref[...] = (dest0 * (ci == 0) + dest1 * (ci == 1)
                    + dp * (ci == 2) + dq * (ci == 3)).astype(jnp.int32)
    rwp_ref[...] = rw0 * (ci == 0) + rw1 * (ci == 1)

    mpos = lax.broadcasted_iota(jnp.int32, (1, 128), 1).astype(jnp.float32) * jnp.float32(TM)
    acc = jnp.full((1, 128), 7, jnp.int32)
    for e in range(E):
        po_e = po[0:1, e:e + 1]
        pc_e = pc[0:1, e:e + 1]
        inside = (mpos >= po_e) & (mpos < po_e + pc_e)
        acc = jnp.where(inside, e, acc)
    te_ref[...] = acc


def _router(xf, Wr, br):
    return pl.pallas_call(
        _router_body,
        out_shape=(
            jax.ShapeDtypeStruct((T, 8), jnp.float32),
            jax.ShapeDtypeStruct((T, 8), jnp.int32),
            jax.ShapeDtypeStruct((1, 128), jnp.int32),
            jax.ShapeDtypeStruct((1, 1), jnp.float32),
        ),
    )(xf, Wr, br.reshape(1, E))


def _sc_scatter(xf, qq, pp, srp, d0, d1, dq, dp, dpr2):
    mesh = plsc.VectorSubcoreMesh(core_axis_name="c", subcore_axis_name="s")

    @functools.partial(
        pl.kernel,
        out_type=[
            jax.ShapeDtypeStruct((PBUF, D), jnp.float32),
            jax.ShapeDtypeStruct((PBUF, D), jnp.float32),
            jax.ShapeDtypeStruct((PBUF, PATCH_K), jnp.float32),
            jax.ShapeDtypeStruct((PBUF, 128), jnp.float32),
        ],
        mesh=mesh,
        scratch_types=[
            pltpu.VMEM((CH, D), jnp.float32),
            pltpu.VMEM((CH, D), jnp.float32),
            pltpu.VMEM((CH, PATCH_K), jnp.float32),
            pltpu.VMEM((2 * CH, 128), jnp.float32),
            pltpu.VMEM((CH,), jnp.int32),
            pltpu.VMEM((CH,), jnp.int32),
            pltpu.VMEM((CH,), jnp.int32),
            pltpu.VMEM((CH,), jnp.int32),
            pltpu.VMEM((2 * CH,), jnp.int32),
            pltpu.SemaphoreType.DMA,
            pltpu.SemaphoreType.DMA,
        ],
    )
    def scat(xf_h, qq_h, pp_h, srp_h, d0_h, d1_h, dq_h, dp_h, dpr_h,
             xs_h, qs_h, ps_h, srs_h,
             bufx, bufq, bufp, bufsr, i0, i1, iq, ip, ipr, semi, sem):
        wid = lax.axis_index("s") * 2 + lax.axis_index("c")
        for k in range(TPW // CH):
            row = (TPW // CH) * wid + k
            base = row * CH
            ci0 = pltpu.async_copy(d0_h.at[row], i0, semi)
            ci1 = pltpu.async_copy(d1_h.at[row], i1, semi)
            ciq = pltpu.async_copy(dq_h.at[row], iq, semi)
            cip = pltpu.async_copy(dp_h.at[row], ip, semi)
            cir = pltpu.async_copy(dpr_h.at[row], ipr, semi)
            cx = pltpu.async_copy(xf_h.at[pl.ds(base, CH)], bufx, semi)
            cq = pltpu.async_copy(qq_h.at[pl.ds(base, CH)], bufq, semi)
            cp = pltpu.async_copy(pp_h.at[pl.ds(base, CH)], bufp, semi)
            cs = pltpu.async_copy(srp_h.at[pl.ds(2 * base, 2 * CH)], bufsr, semi)
            for c in (ci0, ci1, ciq, cip, cir, cx, cq, cp, cs):
                c.wait()
            a = pltpu.async_copy(bufx, xs_h.at[i0], sem)
            b = pltpu.async_copy(bufx, xs_h.at[i1], sem)
            c = pltpu.async_copy(bufq, qs_h.at[iq], sem)
            d = pltpu.async_copy(bufp, ps_h.at[ip], sem)
            e = pltpu.async_copy(bufsr, srs_h.at[ipr], sem)
            for h in (a, b, c, d, e):
                h.wait()

    return scat(xf, qq, pp, srp, d0, d1, dq, dp, dpr2)


def _gmm_body(te_ref, xs_ref, sr_ref, ps_ref, qs_ref,
              w1_ref, wsm_ref, wp_ref, wq_ref, b1_ref, w2_ref, b2_ref,
              out_ref, h_ref):
    m = pl.program_id(0)
    f = pl.program_id(1)
    e = te_ref[m]

    h_ref[...] = (jnp.dot(xs_ref[...].astype(jnp.bfloat16),
                          w1_ref[0].astype(jnp.bfloat16),
                          preferred_element_type=jnp.float32)
                  + jnp.dot(sr_ref[:, 0:16], wsm_ref[0], preferred_element_type=jnp.float32)
                  + b1_ref[0])

    @pl.when(e == 1)
    def _():
        h_ref[...] = h_ref[...] + jnp.dot(ps_ref[...].astype(jnp.bfloat16),
                                          wp_ref[...].astype(jnp.bfloat16),
                                          preferred_element_type=jnp.float32)

    @pl.when(e == 3)
    def _():
        h_ref[...] = h_ref[...] + jnp.dot(qs_ref[...].astype(jnp.bfloat16),
                                          wq_ref[...].astype(jnp.bfloat16),
                                          preferred_element_type=jnp.float32)

    hg = jax.nn.gelu(h_ref[...])
    acc = jnp.dot(hg.astype(jnp.bfloat16), w2_ref[0].astype(jnp.bfloat16),
                  preferred_element_type=jnp.float32)

    @pl.when(f == 0)
    def _():
        out_ref[...] = jnp.zeros_like(out_ref)

    out_ref[...] += acc

    @pl.when(f == NF - 1)
    def _():
        out_ref[...] = (out_ref[...] + b2_ref[0]) * sr_ref[:, 16:17]


def _gmm(te, xs, sr, ps, qs, W1, Wsm, Wp, Wq, b1, W2, b2):
    grid_spec = pltpu.PrefetchScalarGridSpec(
        num_scalar_prefetch=1,
        grid=(NT, NF),
        in_specs=[
            pl.BlockSpec((TM, D), lambda m, f, te: (m, 0)),
            pl.BlockSpec((TM, 128), lambda m, f, te: (m, 0)),
            pl.BlockSpec((TM, PATCH_K), lambda m, f, te: (m, 0)),
            pl.BlockSpec((TM, D), lambda m, f, te: (m, 0)),
            pl.BlockSpec((1, D, FB), lambda m, f, te: (te[m], 0, f)),
            pl.BlockSpec((1, 16, FB), lambda m, f, te: (te[m], 0, f)),
            pl.BlockSpec((PATCH_K, FB), lambda m, f, te: (0, jnp.where(te[m] == 1, f, 0))),
            pl.BlockSpec((D, FB), lambda m, f, te: (0, jnp.where(te[m] == 3, f, 0))),
            pl.BlockSpec((1, 1, FB), lambda m, f, te: (te[m], 0, f)),
            pl.BlockSpec((1, FB, D), lambda m, f, te: (te[m], f, 0)),
            pl.BlockSpec((1, 1, D), lambda m, f, te: (te[m], 0, 0)),
        ],
        out_specs=pl.BlockSpec((TM, D), lambda m, f, te: (m, 0)),
        scratch_shapes=[pltpu.VMEM((TM, FB), jnp.float32)],
    )
    return pl.pallas_call(
        _gmm_body,
        grid_spec=grid_spec,
        out_shape=jax.ShapeDtypeStruct((PBUF, D), jnp.float32),
    )(te, xs, sr, ps, qs, W1, Wsm, Wp, Wq,
      b1.reshape(E, 1, F), W2, b2.reshape(E, 1, D))


def _sc_combine(outs, d0, d1):
    mesh = plsc.VectorSubcoreMesh(core_axis_name="c", subcore_axis_name="s")

    @functools.partial(
        pl.kernel,
        out_type=[
            jax.ShapeDtypeStruct((T, D), jnp.float32),
            jax.ShapeDtypeStruct((T, D), jnp.float32),
        ],
        mesh=mesh,
        scratch_types=[
            pltpu.VMEM((TPW, D), jnp.float32),
            pltpu.VMEM((TPW,), jnp.int32),
            pltpu.SemaphoreType.DMA,
        ],
    )
    def comb(outs_h, d0_h, d1_h, f0_h, f1_h, bufa, i0, sem):
        wid = lax.axis_index("s") * 2 + lax.axis_index("c")
        base = wid * TPW
        pltpu.sync_copy(d0_h.at[wid], i0)
        pltpu.async_copy(outs_h.at[i0], bufa, sem).wait()
        pltpu.sync_copy(bufa, f0_h.at[pl.ds(base, TPW)])
        pltpu.sync_copy(d1_h.at[wid], i0)
        pltpu.async_copy(outs_h.at[i0], bufa, sem).wait()
        pltpu.sync_copy(bufa, f1_h.at[pl.ds(base, TPW)])

    return comb(outs, d0, d1)


def _add_body(a_ref, b_ref, o_ref):
    o_ref[...] = a_ref[...] + b_ref[...]


def _final_add(a, b):
    bm = T // 8
    return pl.pallas_call(
        _add_body,
        grid=(8,),
        in_specs=[pl.BlockSpec((bm, D), lambda i: (i, 0)),
                  pl.BlockSpec((bm, D), lambda i: (i, 0))],
        out_specs=pl.BlockSpec((bm, D), lambda i: (i, 0)),
        out_shape=jax.ShapeDtypeStruct((T, D), jnp.float32),
    )(a, b)


def kernel(x, avg_question_embedding_flat, flow_vectors, raw_patches, frame_deltas,
           Wr, br, W1, b1, W2, b2, Wf, Wp, Wq, Wd):
    xf = x.reshape(T, D)
    pp = raw_patches.reshape(T, PATCH_K)
    qq = avg_question_embedding_flat
    ssm = jnp.concatenate(
        [flow_vectors.reshape(T, 2), frame_deltas.reshape(T, 4),
         jnp.zeros((T, 10), jnp.float32)], axis=1)
    Wsm = jnp.zeros((E, 16, F), jnp.float32)
    Wsm = Wsm.at[0, 0:2].set(Wf)
    Wsm = Wsm.at[4, 2:6].set(Wd)

    rwp, dst, te, aux = _router(xf, Wr, br)

    d0 = dst[:, 0].reshape(T // CH, CH)
    d1 = dst[:, 1].reshape(T // CH, CH)
    dpm = dst[:, 2].reshape(T // CH, CH)
    dqm = dst[:, 3].reshape(T // CH, CH)
    dpr2 = dst[:, 0:2].reshape(T // CH, 2 * CH)
    srp = jnp.concatenate(
        [jnp.repeat(ssm, 2, axis=0),
         rwp[:, 0:2].reshape(2 * T, 1),
         jnp.zeros((2 * T, 111), jnp.float32)], axis=1)

    xs, qs, ps, srs = _sc_scatter(xf, qq, pp, srp, d0, d1, dqm, dpm, dpr2)

    outs = _gmm(te.reshape(128), xs, srs, ps, qs,
                W1, Wsm, Wp, Wq, b1, W2, b2)

    f0, f1 = _sc_combine(outs, dst[:, 0].reshape(NW, TPW), dst[:, 1].reshape(NW, TPW))
    out = _final_add(f0, f1)
    return out.reshape(B, S, D), aux[0, 0]
